# baseline (speedup 1.0000x reference)
import jax
import jax.numpy as jnp
from jax import lax
from jax.experimental import pallas as pl
from jax.experimental.pallas import tpu as pltpu
from jax.experimental.pallas import tpu_sc as plsc

L = 16
NW = 16
N = 20000
NPAD = 20480
NA = NPAD // NW
NV = NA // L
G = 50
GP = 64
JB = 5
HB = 256
NR = 10

POS_T = 0.7
NEG_T = 0.3
TOTAL = 256
MAX_POS = 128
SIG2 = 9.0

_LN_COEFS = (
    -0.0022883228657252968, 0.038030295273843794, -0.2864361250512785,
    1.2917075421662867, -3.8809206183156606, 8.178308102497969,
    -12.396895192830529, 13.666792234339184, -11.06002906824556,
    7.031391849388096, -2.5796606939698807,
)
_LN2 = 0.6931471805599453


def _ln_12(x):
    acc = jnp.full((L,), _LN_COEFS[0], jnp.float32)
    for c in _LN_COEFS[1:]:
        acc = acc * x + jnp.float32(c)
    return acc


def _ln_pos(x):
    bits = plsc.bitcast(x, jnp.int32)
    e = ((bits >> 23) & 0xFF) - 127
    mant = plsc.bitcast((bits & 0x7FFFFF) | 0x3F800000, jnp.float32)
    return e.astype(jnp.float32) * jnp.float32(_LN2) + _ln_12(mant)


def _iota():
    return lax.broadcasted_iota(jnp.int32, (L,), 0)


def _walk(histsum, t):
    def body(c, carry):
        cnt, bacc, sacc = carry
        chunk = histsum[pl.ds(c * L, L)]
        cs = jnp.cumsum(chunk)
        below = cnt + cs - chunk
        is_b = (below < t) & (below + chunk >= t)
        bacc = bacc + jnp.where(is_b, c * L + _iota(), 0)
        sacc = sacc + jnp.where(is_b, below, 0)
        cnt = cnt + jnp.max(cs)
        return cnt, bacc, sacc

    zero = jnp.zeros((L,), jnp.int32)
    _, bacc, sacc = lax.fori_loop(0, HB // L, body,
                                  (jnp.int32(0), zero, zero))
    return jnp.sum(bacc), jnp.sum(sacc)


def _radix_round(r, wid, get_cand, sh, top, flip, pref, t,
                 hist2d, histsum, rdhist, sthist):
    zero = jnp.zeros((L,), jnp.int32)
    ones = jnp.ones((L,), jnp.int32)

    def zbody(i, _):
        hist2d[pl.ds(i * L, L)] = zero
        return 0
    lax.fori_loop(0, HB, zbody, 0)

    io = _iota()

    def scan(v, _):
        mask, key = get_cand(v)
        if not top:
            mask = mask & ((key >> (sh + 8)) == (pref >> (sh + 8)))
        bucket = (key >> sh) & (HB - 1)
        if flip:
            bucket = (HB - 1) - bucket
        slot = io * HB + bucket
        plsc.addupdate_scatter(hist2d, [slot], ones, mask=mask)
        return 0
    lax.fori_loop(0, NV, scan, 0)

    def lred_c(c, _):
        def lred_l(l, acc):
            return acc + hist2d[pl.ds(l * HB + c * L, L)]
        histsum[pl.ds(c * L, L)] = lax.fori_loop(0, L, lred_l, zero)
        return 0
    lax.fori_loop(0, HB // L, lred_c, 0)

    pltpu.sync_copy(histsum, sthist.at[pl.ds(r * NW * HB + wid * HB, HB)])
    plsc.subcore_barrier()
    pltpu.sync_copy(sthist.at[pl.ds(r * NW * HB, NW * HB)], rdhist)

    def gred_c(c, _):
        def gred_i(i, acc):
            return acc + rdhist[pl.ds(i * HB + c * L, L)]
        histsum[pl.ds(c * L, L)] = lax.fori_loop(0, NW, gred_i, zero)
        return 0
    lax.fori_loop(0, HB // L, gred_c, 0)

    bstar, taken = _walk(histsum, t)
    if flip:
        bstar = (HB - 1) - bstar
    return pref | (bstar << sh), t - taken


_GO_Y0 = 0
_GO_X0 = G * L
_GO_Y1 = 2 * G * L
_GO_X1 = 3 * G * L
_GO_AB = 4 * G * L
_GO_C0 = 5 * G * L
_GO_C1 = 5 * G * L + GP
_GO_C2 = 5 * G * L + 2 * GP
_GO_C3 = 5 * G * L + 3 * GP
_GO_HW = 5 * G * L + 4 * GP
GTB = _GO_HW + 2 * L


def _sc_kernel_body(anchh, scrh, prdh, rkh, gtbh,
                    out_hbm,
                    anch, scr, prd, cy0, cx0, cy1, cx1, vrkf,
                    gtb,
                    gtmax, maxiou, bestj, insd, areaa, forced, posm, negm,
                    poskey, hist2d, histsum, rdgt, rdhist, rdsm, wv, outv,
                    st_gtmax, st_cnt, st_hist, st_loss, dsem):
    wid = lax.axis_index("s")
    ostart = wid * NA
    gbase = jnp.minimum(ostart, N - NA)
    fzero = jnp.zeros((L,), jnp.float32)
    io = _iota()

    copies = [
        pltpu.async_copy(anchh.at[pl.ds(gbase * 4, NA * 4)], anch, dsem),
        pltpu.async_copy(scrh.at[pl.ds(gbase * 2, NA * 2)], scr, dsem),
        pltpu.async_copy(prdh.at[pl.ds(gbase * 4, NA * 4)], prd, dsem),
        pltpu.async_copy(rkh.at[pl.ds(gbase, NA)], vrkf, dsem),
        pltpu.async_copy(gtbh, gtb, dsem),
    ]
    for cp in copies:
        cp.wait()

    hvec = gtb[pl.ds(_GO_HW, L)]
    wvec = gtb[pl.ds(_GO_HW + L, L)]

    def coords(o):
        return (cy0[pl.ds(o, L)], cx0[pl.ds(o, L)],
                cy1[pl.ds(o, L)], cx1[pl.ds(o, L)])

    def p05(v, _):
        o = v * L
        i4 = (o + io) * 4
        a0 = plsc.load_gather(anch, [i4])
        a1 = plsc.load_gather(anch, [i4 + 1])
        a2 = plsc.load_gather(anch, [i4 + 2])
        a3 = plsc.load_gather(anch, [i4 + 3])
        cy0[pl.ds(o, L)] = a0
        cx0[pl.ds(o, L)] = a1
        cy1[pl.ds(o, L)] = a2
        cx1[pl.ds(o, L)] = a3
        own = (gbase + o + io) >= ostart
        ins = ((a0 >= 0.0) & (a1 >= 0.0) & (a2 <= hvec) & (a3 <= wvec)
               & own)
        insd[pl.ds(o, L)] = jnp.where(ins, 1.0, 0.0)
        areaa[pl.ds(o, L)] = (a2 - a0) * (a3 - a1)
        maxiou[pl.ds(o, L)] = fzero - 1e30
        bestj[pl.ds(o, L)] = jnp.zeros((L,), jnp.int32)
        forced[pl.ds(o, L)] = fzero
        return 0
    lax.fori_loop(0, NV, p05, 0)

    for b in range(G // JB):
        gd = []
        for jj in range(JB):
            j = b * JB + jj
            gd.append((gtb[pl.ds(_GO_Y0 + j * L, L)],
                       gtb[pl.ds(_GO_X0 + j * L, L)],
                       gtb[pl.ds(_GO_Y1 + j * L, L)],
                       gtb[pl.ds(_GO_X1 + j * L, L)],
                       gtb[pl.ds(_GO_AB + j * L, L)]))

        def p1(v, carry):
            gtm = list(carry[0])
            gti = list(carry[1])
            o = v * L
            a0, a1, a2, a3 = coords(o)
            ins = insd[pl.ds(o, L)] > 0.5
            area_a = areaa[pl.ds(o, L)]
            best = maxiou[pl.ds(o, L)]
            bj = bestj[pl.ds(o, L)]
            oio = o + io
            for jj in range(JB):
                g0, g1, g2, g3, ab = gd[jj]
                ih = jnp.maximum(jnp.minimum(a2, g2) - jnp.maximum(a0, g0),
                                 0.0)
                iw = jnp.maximum(jnp.minimum(a3, g3) - jnp.maximum(a1, g1),
                                 0.0)
                inter = ih * iw
                iou = inter / ((area_a + ab) - inter)
                iou = jnp.where(ins, iou, -1.0)
                upd = iou > best
                best = jnp.where(upd, iou, best)
                bj = jnp.where(upd, b * JB + jj, bj)
                upd2 = iou > gtm[jj]
                gtm[jj] = jnp.where(upd2, iou, gtm[jj])
                gti[jj] = jnp.where(upd2, oio, gti[jj])
            maxiou[pl.ds(o, L)] = best
            bestj[pl.ds(o, L)] = bj
            return tuple(gtm), tuple(gti)

        init = (tuple(fzero - 1e30 for _ in range(JB)),
                tuple(jnp.zeros((L,), jnp.int32) for _ in range(JB)))
        gtm, gti = lax.fori_loop(0, NV, p1, init)
        for jj in range(JB):
            j = b * JB + jj
            gtmax[pl.ds(j * L, L)] = gtm[jj]
            rdgt[pl.ds(j * L, L)] = gti[jj].astype(jnp.float32)

    pltpu.sync_copy(gtmax, st_gtmax.at[pl.ds(wid * G * L, G * L)])
    plsc.subcore_barrier()
    pltpu.sync_copy(st_gtmax, rdgt.at[pl.ds(G * L, NW * G * L)])

    onesf = fzero + 1.0
    for j in range(G):
        def fmax(i, acc):
            return jnp.maximum(
                acc, rdgt[pl.ds(G * L + i * G * L + j * L, L)])
        gm = lax.fori_loop(0, NW, fmax, fzero - 1e30)
        g = jnp.max(gm)
        mine = gtmax[pl.ds(j * L, L)]
        match = (mine >= g) & (mine > 0.0)
        cidx = rdgt[pl.ds(j * L, L)].astype(jnp.int32)
        plsc.store_scatter(forced, [cidx], onesf, mask=match)

    def p2_body(v, carry):
        cp_acc, cn_acc = carry
        o = v * L
        best = maxiou[pl.ds(o, L)]
        ins = insd[pl.ds(o, L)] > 0.5
        fc = forced[pl.ds(o, L)] > 0.5
        pm = ins & ((best >= POS_T) | fc)
        nm = ins & (best < NEG_T) & (best >= 0.0)
        posm[pl.ds(o, L)] = jnp.where(pm, 1.0, 0.0)
        negm[pl.ds(o, L)] = jnp.where(nm, 1.0, 0.0)
        key = plsc.bitcast(best, jnp.int32)
        poskey[pl.ds(o, L)] = jnp.where(pm, key, 0)
        return (cp_acc + jnp.where(pm, 1.0, 0.0),
                cn_acc + jnp.where(nm, 1.0, 0.0))

    cp_acc, cn_acc = lax.fori_loop(0, NV, p2_body, (fzero, fzero))
    wv[pl.ds(0, L)] = cp_acc
    wv[pl.ds(L, L)] = cn_acc
    pltpu.sync_copy(wv, st_cnt.at[pl.ds(wid * 2 * L, 2 * L)])
    plsc.subcore_barrier()
    pltpu.sync_copy(st_cnt, rdsm)

    def cmerge(i, carry):
        a, b2 = carry
        return (a + rdsm[pl.ds(i * 2 * L, L)],
                b2 + rdsm[pl.ds(i * 2 * L + L, L)])
    cpv, cnv = lax.fori_loop(0, NW, cmerge, (fzero, fzero))
    cnt_pos = jnp.sum(cpv)
    cnt_neg = jnp.sum(cnv)
    n_pos = jnp.minimum(cnt_pos, float(MAX_POS))
    need_f = float(TOTAL) - n_pos
    cp_i = cnt_pos.astype(jnp.int32)
    cn_i = cnt_neg.astype(jnp.int32)
    need_i = jnp.int32(TOTAL) - jnp.minimum(cp_i, MAX_POS)
    kfill_i = jnp.maximum(need_i - cn_i, 0)
    pos_over = cnt_pos > float(MAX_POS)
    neg_over = cnt_neg > need_f

    def get_pos(v):
        o = v * L
        return posm[pl.ds(o, L)] > 0.5, poskey[pl.ds(o, L)]

    def get_neg(v):
        o = v * L
        return (negm[pl.ds(o, L)] > 0.5,
                plsc.bitcast(vrkf[pl.ds(o, L)], jnp.int32))

    def rrnd(r, get, sh, top, flip, pref, t):
        return _radix_round(r, wid, get, sh, top, flip, pref, t,
                            hist2d, histsum, rdhist, st_hist)

    def pos_rounds(_):
        pref, t = rrnd(0, get_pos, 24, True, True, jnp.int32(0),
                       jnp.int32(MAX_POS))
        pref, t = rrnd(1, get_pos, 16, False, True, pref, t)
        pref, t = rrnd(2, get_pos, 8, False, True, pref, t)
        pref, t = rrnd(3, get_pos, 0, False, True, pref, t)
        kv_, trem = pref, t

        def get_tie(v):
            o = v * L
            m = (posm[pl.ds(o, L)] > 0.5) & (poskey[pl.ds(o, L)] == kv_)
            return m, gbase + o + io

        pref2, t2 = rrnd(4, get_tie, 8, True, False, jnp.int32(0), trem)
        pref2, _ = rrnd(5, get_tie, 0, False, False, pref2, t2)
        return kv_, pref2

    kv, ki = lax.cond(pos_over, pos_rounds,
                      lambda _: (jnp.int32(0), jnp.int32(NPAD)),
                      0)

    def neg_rounds(_):
        pref3, t3 = rrnd(6, get_neg, 8, True, False, jnp.int32(0), need_i)
        pref3, _ = rrnd(7, get_neg, 0, False, False, pref3, t3)
        return pref3

    kr = lax.cond(neg_over, neg_rounds, lambda _: jnp.int32(NPAD + 1), 0)

    def get_fill(v):
        o = v * L
        idx = gbase + o + io
        m = (negm[pl.ds(o, L)] <= 0.5) & (idx >= ostart)
        return m, idx

    def fill_rounds(_):
        pref4, t4 = rrnd(8, get_fill, 8, True, False, jnp.int32(0), kfill_i)
        pref4, _ = rrnd(9, get_fill, 0, False, False, pref4, t4)
        return pref4

    kf = lax.cond(kfill_i > 0, fill_rounds, lambda _: jnp.int32(-1), 0)

    def p4_body(v, carry):
        acc_cls, acc_reg = carry
        o = v * L
        idx = gbase + o + io
        pm = posm[pl.ds(o, L)] > 0.5
        nm = negm[pl.ds(o, L)] > 0.5
        key = poskey[pl.ds(o, L)]
        rk = plsc.bitcast(vrkf[pl.ds(o, L)], jnp.int32)
        sel_pos = pm & ((key > kv) | ((key == kv) & (idx <= ki)))
        sel_neg = nm & (rk <= kr)
        sel_fill = (~nm) & (idx >= ostart) & (idx <= kf)
        neg_w = sel_neg | sel_fill

        i2 = (o + io) * 2
        sv0 = plsc.load_gather(scr, [i2])
        sv1 = plsc.load_gather(scr, [i2 + 1])
        m = jnp.maximum(sv0, sv1)
        esum = jnp.exp(sv0 - m) + jnp.exp(sv1 - m)
        lse = m + _ln_12(esum)
        lp0 = sv0 - lse
        lp1 = sv1 - lse
        acc_cls = (acc_cls - jnp.where(sel_pos, lp1, 0.0)
                   - jnp.where(neg_w, lp0, 0.0))

        a0, a1, a2, a3 = coords(o)
        bj = bestj[pl.ds(o, L)]
        g0 = plsc.load_gather(gtb, [_GO_C0 + bj])
        g1 = plsc.load_gather(gtb, [_GO_C1 + bj])
        g2 = plsc.load_gather(gtb, [_GO_C2 + bj])
        g3 = plsc.load_gather(gtb, [_GO_C3 + bj])
        a_h = a2 - a0
        a_w = a3 - a1
        a_cy = a0 + 0.5 * a_h
        a_cx = a1 + 0.5 * a_w
        g_h = g2 - g0
        g_w = g3 - g1
        g_cy = g0 + 0.5 * g_h
        g_cx = g1 + 0.5 * g_w
        eps = jnp.float32(1e-8)
        ty = (g_cy - a_cy) / (a_h + eps)
        tx = (g_cx - a_cx) / (a_w + eps)
        th = _ln_pos(jnp.maximum(g_h, eps)) - _ln_pos(jnp.maximum(a_h, eps))
        tw = _ln_pos(jnp.maximum(g_w, eps)) - _ln_pos(jnp.maximum(a_w, eps))
        i4 = (o + io) * 4
        ssum = fzero
        for c, tgt in ((0, tx), (1, ty), (2, tw), (3, th)):
            d = plsc.load_gather(prd, [i4 + c]) - tgt
            ad = jnp.abs(d)
            sl = jnp.where(ad < 1.0 / SIG2, 0.5 * SIG2 * d * d,
                           ad - 0.5 / SIG2)
            ssum = ssum + sl
        acc_reg = acc_reg + jnp.where(sel_pos, ssum, 0.0)
        return acc_cls, acc_reg

    acc_cls, acc_reg = lax.fori_loop(0, NV, p4_body, (fzero, fzero))
    wv[pl.ds(0, L)] = acc_cls
    wv[pl.ds(L, L)] = acc_reg
    pltpu.sync_copy(wv, st_loss.at[pl.ds(wid * 2 * L, 2 * L)])
    plsc.subcore_barrier()
    pltpu.sync_copy(st_loss, rdsm)

    def lmerge(i, carry):
        a, b2 = carry
        return (a + rdsm[pl.ds(i * 2 * L, L)],
                b2 + rdsm[pl.ds(i * 2 * L + L, L)])
    av, bv = lax.fori_loop(0, NW, lmerge, (fzero, fzero))
    cls_sum = jnp.sum(av)
    reg_sum = jnp.sum(bv)
    numer = (jnp.where(io == 0, cls_sum, 0.0)
             + jnp.where(io == 1, reg_sum, 0.0))
    denom = jnp.where(io == 1, jnp.maximum(n_pos, 1.0),
                      jnp.float32(TOTAL))
    outv[pl.ds(0, L)] = numer / denom

    @pl.when(wid == 0)
    def _():
        pltpu.sync_copy(outv, out_hbm)


def _build_call():
    mesh = plsc.VectorSubcoreMesh(core_axis_name="c", subcore_axis_name="s",
                                  num_cores=1, num_subcores=NW)
    f32, i32 = jnp.float32, jnp.int32
    return pl.kernel(
        _sc_kernel_body,
        out_type=[
            jax.ShapeDtypeStruct((L,), f32),
        ],
        mesh=mesh,
        compiler_params=pltpu.CompilerParams(needs_layout_passes=False),
        scratch_types=[
            pltpu.VMEM((NA * 4,), f32),
            pltpu.VMEM((NA * 2,), f32),
            pltpu.VMEM((NA * 4,), f32),
            pltpu.VMEM((NA,), f32),
            pltpu.VMEM((NA,), f32),
            pltpu.VMEM((NA,), f32),
            pltpu.VMEM((NA,), f32),
            pltpu.VMEM((NA,), f32),
            pltpu.VMEM((GTB,), f32),
            pltpu.VMEM((G * L,), f32),
            pltpu.VMEM((NA,), f32),
            pltpu.VMEM((NA,), i32),
            pltpu.VMEM((NA,), f32),
            pltpu.VMEM((NA,), f32),
            pltpu.VMEM((NA,), f32),
            pltpu.VMEM((NA,), f32),
            pltpu.VMEM((NA,), f32),
            pltpu.VMEM((NA,), i32),
            pltpu.VMEM((L * HB,), i32),
            pltpu.VMEM((HB,), i32),
            pltpu.VMEM(((NW + 1) * G * L,), f32),
            pltpu.VMEM((NW * HB,), i32),
            pltpu.VMEM((NW * 2 * L,), f32),
            pltpu.VMEM((2 * L,), f32),
            pltpu.VMEM((L,), f32),
            pltpu.VMEM_SHARED((NW * G * L,), f32),
            pltpu.VMEM_SHARED((NW * 2 * L,), f32),
            pltpu.VMEM_SHARED((NR * NW * HB,), i32),
            pltpu.VMEM_SHARED((NW * 2 * L,), f32),
            pltpu.SemaphoreType.DMA,
        ],
    )


def kernel(image_shape, anchors, rpn_score, rpn_bboxes_txtytwth, gt_bboxes):
    f32 = jnp.float32
    with jax.ensure_compile_time_eval():
        rngv = jax.random.uniform(jax.random.key(123), (N,))
        order = jnp.argsort(-rngv, stable=True)
        rank = jnp.argsort(order, stable=True).astype(jnp.int32)
        rkf = lax.bitcast_convert_type(rank, f32)

    af = anchors.astype(f32).reshape(-1)
    sf = rpn_score.astype(f32).reshape(-1)
    pf = rpn_bboxes_txtytwth.astype(f32).reshape(-1)

    gt = gt_bboxes.astype(f32)
    ab = ((gt[:, 2] - gt[:, 0]) * (gt[:, 3] - gt[:, 1]) + 1e-9)
    gpad = jnp.zeros((GP - G,), f32)
    gtbuf = jnp.concatenate([
        jnp.broadcast_to(gt[:, 0:1], (G, L)).reshape(-1),
        jnp.broadcast_to(gt[:, 1:2], (G, L)).reshape(-1),
        jnp.broadcast_to(gt[:, 2:3], (G, L)).reshape(-1),
        jnp.broadcast_to(gt[:, 3:4], (G, L)).reshape(-1),
        jnp.broadcast_to(ab[:, None], (G, L)).reshape(-1),
        gt[:, 0], gpad, gt[:, 1], gpad, gt[:, 2], gpad, gt[:, 3], gpad,
        jnp.full((L,), image_shape[0], f32),
        jnp.full((L,), image_shape[1], f32),
    ])

    call = _build_call()
    out = call(af, sf, pf, rkf, gtbuf)[0]
    return (out[0], out[1])


# speedup vs baseline: 1.6554x; 1.6554x over previous
import jax
import jax.numpy as jnp
from jax import lax
from jax.experimental import pallas as pl
from jax.experimental.pallas import tpu as pltpu
from jax.experimental.pallas import tpu_sc as plsc

L = 16
NW = 16
N = 20000
NPAD = 20480
NA = NPAD // NW
NV = NA // L
G = 50
GP = 64
JB = 5
HB = 256
NR = 10

POS_T = 0.7
NEG_T = 0.3
TOTAL = 256
MAX_POS = 128
SIG2 = 9.0

_LN_COEFS = (
    -0.0022883228657252968, 0.038030295273843794, -0.2864361250512785,
    1.2917075421662867, -3.8809206183156606, 8.178308102497969,
    -12.396895192830529, 13.666792234339184, -11.06002906824556,
    7.031391849388096, -2.5796606939698807,
)
_LN2 = 0.6931471805599453


def _ln_12(x):
    acc = jnp.full((L,), _LN_COEFS[0], jnp.float32)
    for c in _LN_COEFS[1:]:
        acc = acc * x + jnp.float32(c)
    return acc


def _ln_pos(x):
    bits = plsc.bitcast(x, jnp.int32)
    e = ((bits >> 23) & 0xFF) - 127
    mant = plsc.bitcast((bits & 0x7FFFFF) | 0x3F800000, jnp.float32)
    return e.astype(jnp.float32) * jnp.float32(_LN2) + _ln_12(mant)


def _iota():
    return lax.broadcasted_iota(jnp.int32, (L,), 0)


def _walk(histsum, t):
    def body(c, carry):
        cnt, bacc, sacc = carry
        chunk = histsum[pl.ds(c * L, L)]
        cs = jnp.cumsum(chunk)
        below = cnt + cs - chunk
        is_b = (below < t) & (below + chunk >= t)
        bacc = bacc + jnp.where(is_b, c * L + _iota(), 0)
        sacc = sacc + jnp.where(is_b, below, 0)
        cnt = cnt + jnp.max(cs)
        return cnt, bacc, sacc

    zero = jnp.zeros((L,), jnp.int32)
    _, bacc, sacc = lax.fori_loop(0, HB // L, body,
                                  (jnp.int32(0), zero, zero))
    return jnp.sum(bacc), jnp.sum(sacc)


def _radix_round(r, wid, get_cand, sh, top, flip, pref, t,
                 hist2d, histsum, rdhist, sthist):
    zero = jnp.zeros((L,), jnp.int32)
    ones = jnp.ones((L,), jnp.int32)

    def zbody(i, _):
        hist2d[pl.ds(i * L, L)] = zero
        return 0
    lax.fori_loop(0, HB, zbody, 0)

    io = _iota()

    def scan(v, _):
        mask, key = get_cand(v)
        if not top:
            mask = mask & ((key >> (sh + 8)) == (pref >> (sh + 8)))
        bucket = (key >> sh) & (HB - 1)
        if flip:
            bucket = (HB - 1) - bucket
        slot = io * HB + bucket
        plsc.addupdate_scatter(hist2d, [slot], ones, mask=mask)
        return 0
    lax.fori_loop(0, NV, scan, 0)

    def lred_c(c, _):
        def lred_l(l, acc):
            return acc + hist2d[pl.ds(l * HB + c * L, L)]
        histsum[pl.ds(c * L, L)] = lax.fori_loop(0, L, lred_l, zero)
        return 0
    lax.fori_loop(0, HB // L, lred_c, 0)

    pltpu.sync_copy(histsum, sthist.at[pl.ds(r * NW * HB + wid * HB, HB)])
    plsc.subcore_barrier()
    pltpu.sync_copy(sthist.at[pl.ds(r * NW * HB, NW * HB)], rdhist)

    def gred_c(c, _):
        def gred_i(i, acc):
            return acc + rdhist[pl.ds(i * HB + c * L, L)]
        histsum[pl.ds(c * L, L)] = lax.fori_loop(0, NW, gred_i, zero)
        return 0
    lax.fori_loop(0, HB // L, gred_c, 0)

    bstar, taken = _walk(histsum, t)
    if flip:
        bstar = (HB - 1) - bstar
    return pref | (bstar << sh), t - taken


_GO_Y0 = 0
_GO_X0 = G * L
_GO_Y1 = 2 * G * L
_GO_X1 = 3 * G * L
_GO_AB = 4 * G * L
_GO_C0 = 5 * G * L
_GO_C1 = 5 * G * L + GP
_GO_C2 = 5 * G * L + 2 * GP
_GO_C3 = 5 * G * L + 3 * GP
_GO_HW = 5 * G * L + 4 * GP
GTB = _GO_HW + 2 * L


def _sc_kernel_body(ay0h, ax0h, ay1h, ax1h, s0h, s1h, p0h, p1h, p2h, p3h,
                    rkh, gtbh,
                    out_hbm,
                    cy0, cx0, cy1, cx1, vs0, vs1, vp0, vp1, vp2, vp3, vrkf,
                    gtb,
                    gtmax, maxiou, bestj, insd, areaa, forced, posm, negm,
                    poskey, hist2d, histsum, rdgt, rdhist, rdsm, wv, outv,
                    st_gtmax, st_cnt, st_hist, st_loss, dsem):
    wid = lax.axis_index("s")
    ostart = wid * NA
    gbase = jnp.minimum(ostart, N - NA)
    fzero = jnp.zeros((L,), jnp.float32)
    io = _iota()

    srcs = [ay0h, ax0h, ay1h, ax1h, s0h, s1h, p0h, p1h, p2h, p3h, rkh]
    dsts = [cy0, cx0, cy1, cx1, vs0, vs1, vp0, vp1, vp2, vp3, vrkf]
    copies = [
        pltpu.async_copy(src.at[pl.ds(gbase, NA)], dst, dsem)
        for src, dst in zip(srcs, dsts)
    ]
    copies.append(pltpu.async_copy(gtbh, gtb, dsem))
    for cp in copies:
        cp.wait()

    hvec = gtb[pl.ds(_GO_HW, L)]
    wvec = gtb[pl.ds(_GO_HW + L, L)]

    def coords(o):
        return (cy0[pl.ds(o, L)], cx0[pl.ds(o, L)],
                cy1[pl.ds(o, L)], cx1[pl.ds(o, L)])

    def p05(v, _):
        o = v * L
        a0, a1, a2, a3 = coords(o)
        own = (gbase + o + io) >= ostart
        ins = ((a0 >= 0.0) & (a1 >= 0.0) & (a2 <= hvec) & (a3 <= wvec)
               & own)
        insd[pl.ds(o, L)] = jnp.where(ins, 1.0, 0.0)
        areaa[pl.ds(o, L)] = (a2 - a0) * (a3 - a1)
        maxiou[pl.ds(o, L)] = fzero - 1e30
        bestj[pl.ds(o, L)] = jnp.zeros((L,), jnp.int32)
        forced[pl.ds(o, L)] = fzero
        return 0
    lax.fori_loop(0, NV, p05, 0)

    for b in range(G // JB):
        gd = []
        for jj in range(JB):
            j = b * JB + jj
            gd.append((gtb[pl.ds(_GO_Y0 + j * L, L)],
                       gtb[pl.ds(_GO_X0 + j * L, L)],
                       gtb[pl.ds(_GO_Y1 + j * L, L)],
                       gtb[pl.ds(_GO_X1 + j * L, L)],
                       gtb[pl.ds(_GO_AB + j * L, L)]))

        def p1(v, carry):
            gtm = list(carry[0])
            gti = list(carry[1])
            o = v * L
            a0, a1, a2, a3 = coords(o)
            ins = insd[pl.ds(o, L)] > 0.5
            area_a = areaa[pl.ds(o, L)]
            best = maxiou[pl.ds(o, L)]
            bj = bestj[pl.ds(o, L)]
            oio = o + io
            for jj in range(JB):
                g0, g1, g2, g3, ab = gd[jj]
                ih = jnp.maximum(jnp.minimum(a2, g2) - jnp.maximum(a0, g0),
                                 0.0)
                iw = jnp.maximum(jnp.minimum(a3, g3) - jnp.maximum(a1, g1),
                                 0.0)
                inter = ih * iw
                iou = inter / ((area_a + ab) - inter)
                iou = jnp.where(ins, iou, -1.0)
                upd = iou > best
                best = jnp.where(upd, iou, best)
                bj = jnp.where(upd, b * JB + jj, bj)
                upd2 = iou > gtm[jj]
                gtm[jj] = jnp.where(upd2, iou, gtm[jj])
                gti[jj] = jnp.where(upd2, oio, gti[jj])
            maxiou[pl.ds(o, L)] = best
            bestj[pl.ds(o, L)] = bj
            return tuple(gtm), tuple(gti)

        init = (tuple(fzero - 1e30 for _ in range(JB)),
                tuple(jnp.zeros((L,), jnp.int32) for _ in range(JB)))
        gtm, gti = lax.fori_loop(0, NV, p1, init)
        for jj in range(JB):
            j = b * JB + jj
            gtmax[pl.ds(j * L, L)] = gtm[jj]
            rdgt[pl.ds(j * L, L)] = gti[jj].astype(jnp.float32)

    pltpu.sync_copy(gtmax, st_gtmax.at[pl.ds(wid * G * L, G * L)])
    plsc.subcore_barrier()
    pltpu.sync_copy(st_gtmax, rdgt.at[pl.ds(G * L, NW * G * L)])

    onesf = fzero + 1.0
    for j in range(G):
        def fmax(i, acc):
            return jnp.maximum(
                acc, rdgt[pl.ds(G * L + i * G * L + j * L, L)])
        gm = lax.fori_loop(0, NW, fmax, fzero - 1e30)
        g = jnp.max(gm)
        mine = gtmax[pl.ds(j * L, L)]
        match = (mine >= g) & (mine > 0.0)
        cidx = rdgt[pl.ds(j * L, L)].astype(jnp.int32)
        plsc.store_scatter(forced, [cidx], onesf, mask=match)

    def p2_body(v, carry):
        cp_acc, cn_acc = carry
        o = v * L
        best = maxiou[pl.ds(o, L)]
        ins = insd[pl.ds(o, L)] > 0.5
        fc = forced[pl.ds(o, L)] > 0.5
        pm = ins & ((best >= POS_T) | fc)
        nm = ins & (best < NEG_T) & (best >= 0.0)
        posm[pl.ds(o, L)] = jnp.where(pm, 1.0, 0.0)
        negm[pl.ds(o, L)] = jnp.where(nm, 1.0, 0.0)
        key = plsc.bitcast(best, jnp.int32)
        poskey[pl.ds(o, L)] = jnp.where(pm, key, 0)
        return (cp_acc + jnp.where(pm, 1.0, 0.0),
                cn_acc + jnp.where(nm, 1.0, 0.0))

    cp_acc, cn_acc = lax.fori_loop(0, NV, p2_body, (fzero, fzero))
    wv[pl.ds(0, L)] = cp_acc
    wv[pl.ds(L, L)] = cn_acc
    pltpu.sync_copy(wv, st_cnt.at[pl.ds(wid * 2 * L, 2 * L)])
    plsc.subcore_barrier()
    pltpu.sync_copy(st_cnt, rdsm)

    def cmerge(i, carry):
        a, b2 = carry
        return (a + rdsm[pl.ds(i * 2 * L, L)],
                b2 + rdsm[pl.ds(i * 2 * L + L, L)])
    cpv, cnv = lax.fori_loop(0, NW, cmerge, (fzero, fzero))
    cnt_pos = jnp.sum(cpv)
    cnt_neg = jnp.sum(cnv)
    n_pos = jnp.minimum(cnt_pos, float(MAX_POS))
    need_f = float(TOTAL) - n_pos
    cp_i = cnt_pos.astype(jnp.int32)
    cn_i = cnt_neg.astype(jnp.int32)
    need_i = jnp.int32(TOTAL) - jnp.minimum(cp_i, MAX_POS)
    kfill_i = jnp.maximum(need_i - cn_i, 0)
    pos_over = cnt_pos > float(MAX_POS)
    neg_over = cnt_neg > need_f

    def get_pos(v):
        o = v * L
        return posm[pl.ds(o, L)] > 0.5, poskey[pl.ds(o, L)]

    def get_neg(v):
        o = v * L
        return (negm[pl.ds(o, L)] > 0.5,
                plsc.bitcast(vrkf[pl.ds(o, L)], jnp.int32))

    def rrnd(r, get, sh, top, flip, pref, t):
        return _radix_round(r, wid, get, sh, top, flip, pref, t,
                            hist2d, histsum, rdhist, st_hist)

    def pos_rounds(_):
        pref, t = rrnd(0, get_pos, 24, True, True, jnp.int32(0),
                       jnp.int32(MAX_POS))
        pref, t = rrnd(1, get_pos, 16, False, True, pref, t)
        pref, t = rrnd(2, get_pos, 8, False, True, pref, t)
        pref, t = rrnd(3, get_pos, 0, False, True, pref, t)
        kv_, trem = pref, t

        def get_tie(v):
            o = v * L
            m = (posm[pl.ds(o, L)] > 0.5) & (poskey[pl.ds(o, L)] == kv_)
            return m, gbase + o + io

        pref2, t2 = rrnd(4, get_tie, 8, True, False, jnp.int32(0), trem)
        pref2, _ = rrnd(5, get_tie, 0, False, False, pref2, t2)
        return kv_, pref2

    kv, ki = lax.cond(pos_over, pos_rounds,
                      lambda _: (jnp.int32(0), jnp.int32(NPAD)),
                      0)

    def neg_rounds(_):
        pref3, t3 = rrnd(6, get_neg, 8, True, False, jnp.int32(0), need_i)
        pref3, _ = rrnd(7, get_neg, 0, False, False, pref3, t3)
        return pref3

    kr = lax.cond(neg_over, neg_rounds, lambda _: jnp.int32(NPAD + 1), 0)

    def get_fill(v):
        o = v * L
        idx = gbase + o + io
        m = (negm[pl.ds(o, L)] <= 0.5) & (idx >= ostart)
        return m, idx

    def fill_rounds(_):
        pref4, t4 = rrnd(8, get_fill, 8, True, False, jnp.int32(0), kfill_i)
        pref4, _ = rrnd(9, get_fill, 0, False, False, pref4, t4)
        return pref4

    kf = lax.cond(kfill_i > 0, fill_rounds, lambda _: jnp.int32(-1), 0)

    def p4_body(v, carry):
        acc_cls, acc_reg = carry
        o = v * L
        idx = gbase + o + io
        pm = posm[pl.ds(o, L)] > 0.5
        nm = negm[pl.ds(o, L)] > 0.5
        key = poskey[pl.ds(o, L)]
        rk = plsc.bitcast(vrkf[pl.ds(o, L)], jnp.int32)
        sel_pos = pm & ((key > kv) | ((key == kv) & (idx <= ki)))
        sel_neg = nm & (rk <= kr)
        sel_fill = (~nm) & (idx >= ostart) & (idx <= kf)
        neg_w = sel_neg | sel_fill

        sv0 = vs0[pl.ds(o, L)]
        sv1 = vs1[pl.ds(o, L)]
        m = jnp.maximum(sv0, sv1)
        esum = jnp.exp(sv0 - m) + jnp.exp(sv1 - m)
        lse = m + _ln_12(esum)
        lp0 = sv0 - lse
        lp1 = sv1 - lse
        acc_cls = (acc_cls - jnp.where(sel_pos, lp1, 0.0)
                   - jnp.where(neg_w, lp0, 0.0))

        a0, a1, a2, a3 = coords(o)
        bj = bestj[pl.ds(o, L)]
        g0 = plsc.load_gather(gtb, [_GO_C0 + bj])
        g1 = plsc.load_gather(gtb, [_GO_C1 + bj])
        g2 = plsc.load_gather(gtb, [_GO_C2 + bj])
        g3 = plsc.load_gather(gtb, [_GO_C3 + bj])
        a_h = a2 - a0
        a_w = a3 - a1
        a_cy = a0 + 0.5 * a_h
        a_cx = a1 + 0.5 * a_w
        g_h = g2 - g0
        g_w = g3 - g1
        g_cy = g0 + 0.5 * g_h
        g_cx = g1 + 0.5 * g_w
        eps = jnp.float32(1e-8)
        ty = (g_cy - a_cy) / (a_h + eps)
        tx = (g_cx - a_cx) / (a_w + eps)
        th = _ln_pos(jnp.maximum(g_h, eps)) - _ln_pos(jnp.maximum(a_h, eps))
        tw = _ln_pos(jnp.maximum(g_w, eps)) - _ln_pos(jnp.maximum(a_w, eps))
        ssum = fzero
        for pref_ref, tgt in ((vp0, tx), (vp1, ty), (vp2, tw), (vp3, th)):
            d = pref_ref[pl.ds(o, L)] - tgt
            ad = jnp.abs(d)
            sl = jnp.where(ad < 1.0 / SIG2, 0.5 * SIG2 * d * d,
                           ad - 0.5 / SIG2)
            ssum = ssum + sl
        acc_reg = acc_reg + jnp.where(sel_pos, ssum, 0.0)
        return acc_cls, acc_reg

    acc_cls, acc_reg = lax.fori_loop(0, NV, p4_body, (fzero, fzero))
    wv[pl.ds(0, L)] = acc_cls
    wv[pl.ds(L, L)] = acc_reg
    pltpu.sync_copy(wv, st_loss.at[pl.ds(wid * 2 * L, 2 * L)])
    plsc.subcore_barrier()
    pltpu.sync_copy(st_loss, rdsm)

    def lmerge(i, carry):
        a, b2 = carry
        return (a + rdsm[pl.ds(i * 2 * L, L)],
                b2 + rdsm[pl.ds(i * 2 * L + L, L)])
    av, bv = lax.fori_loop(0, NW, lmerge, (fzero, fzero))
    cls_sum = jnp.sum(av)
    reg_sum = jnp.sum(bv)
    numer = (jnp.where(io == 0, cls_sum, 0.0)
             + jnp.where(io == 1, reg_sum, 0.0))
    denom = jnp.where(io == 1, jnp.maximum(n_pos, 1.0),
                      jnp.float32(TOTAL))
    outv[pl.ds(0, L)] = numer / denom

    @pl.when(wid == 0)
    def _():
        pltpu.sync_copy(outv, out_hbm)


def _build_call():
    mesh = plsc.VectorSubcoreMesh(core_axis_name="c", subcore_axis_name="s",
                                  num_cores=1, num_subcores=NW)
    f32, i32 = jnp.float32, jnp.int32
    return pl.kernel(
        _sc_kernel_body,
        out_type=[
            jax.ShapeDtypeStruct((L,), f32),
        ],
        mesh=mesh,
        compiler_params=pltpu.CompilerParams(needs_layout_passes=False),
        scratch_types=[
            pltpu.VMEM((NA,), f32),
            pltpu.VMEM((NA,), f32),
            pltpu.VMEM((NA,), f32),
            pltpu.VMEM((NA,), f32),
            pltpu.VMEM((NA,), f32),
            pltpu.VMEM((NA,), f32),
            pltpu.VMEM((NA,), f32),
            pltpu.VMEM((NA,), f32),
            pltpu.VMEM((NA,), f32),
            pltpu.VMEM((NA,), f32),
            pltpu.VMEM((NA,), f32),
            pltpu.VMEM((GTB,), f32),
            pltpu.VMEM((G * L,), f32),
            pltpu.VMEM((NA,), f32),
            pltpu.VMEM((NA,), i32),
            pltpu.VMEM((NA,), f32),
            pltpu.VMEM((NA,), f32),
            pltpu.VMEM((NA,), f32),
            pltpu.VMEM((NA,), f32),
            pltpu.VMEM((NA,), f32),
            pltpu.VMEM((NA,), i32),
            pltpu.VMEM((L * HB,), i32),
            pltpu.VMEM((HB,), i32),
            pltpu.VMEM(((NW + 1) * G * L,), f32),
            pltpu.VMEM((NW * HB,), i32),
            pltpu.VMEM((NW * 2 * L,), f32),
            pltpu.VMEM((2 * L,), f32),
            pltpu.VMEM((L,), f32),
            pltpu.VMEM_SHARED((NW * G * L,), f32),
            pltpu.VMEM_SHARED((NW * 2 * L,), f32),
            pltpu.VMEM_SHARED((NR * NW * HB,), i32),
            pltpu.VMEM_SHARED((NW * 2 * L,), f32),
            pltpu.SemaphoreType.DMA,
        ],
    )


def kernel(image_shape, anchors, rpn_score, rpn_bboxes_txtytwth, gt_bboxes):
    f32 = jnp.float32
    with jax.ensure_compile_time_eval():
        rngv = jax.random.uniform(jax.random.key(123), (N,))
        order = jnp.argsort(-rngv, stable=True)
        rank = jnp.argsort(order, stable=True).astype(jnp.int32)
        rkf = lax.bitcast_convert_type(rank, f32)

    a = anchors.astype(f32)
    s = rpn_score.astype(f32)
    p = rpn_bboxes_txtytwth.astype(f32)

    gt = gt_bboxes.astype(f32)
    ab = ((gt[:, 2] - gt[:, 0]) * (gt[:, 3] - gt[:, 1]) + 1e-9)
    gpad = jnp.zeros((GP - G,), f32)
    gtbuf = jnp.concatenate([
        jnp.broadcast_to(gt[:, 0:1], (G, L)).reshape(-1),
        jnp.broadcast_to(gt[:, 1:2], (G, L)).reshape(-1),
        jnp.broadcast_to(gt[:, 2:3], (G, L)).reshape(-1),
        jnp.broadcast_to(gt[:, 3:4], (G, L)).reshape(-1),
        jnp.broadcast_to(ab[:, None], (G, L)).reshape(-1),
        gt[:, 0], gpad, gt[:, 1], gpad, gt[:, 2], gpad, gt[:, 3], gpad,
        jnp.full((L,), image_shape[0], f32),
        jnp.full((L,), image_shape[1], f32),
    ])

    call = _build_call()
    out = call(a[:, 0], a[:, 1], a[:, 2], a[:, 3], s[:, 0], s[:, 1],
               p[:, 0], p[:, 1], p[:, 2], p[:, 3], rkf, gtbuf)[0]
    return (out[0], out[1])


# speedup vs baseline: 1.6835x; 1.0170x over previous
import jax
import jax.numpy as jnp
from jax import lax
from jax.experimental import pallas as pl
from jax.experimental.pallas import tpu as pltpu
from jax.experimental.pallas import tpu_sc as plsc

L = 16
NW = 16
N = 20000
NPAD = 20480
NA = NPAD // NW
NV = NA // L
G = 50
GP = 64
JB = 5
HB = 256
NR = 10

POS_T = 0.7
NEG_T = 0.3
TOTAL = 256
MAX_POS = 128
SIG2 = 9.0

_LN_COEFS = (
    0.010119886147827535, -0.12346802711149933, 0.6590630061074665,
    -2.020333728795536, 3.932845795708761, -5.12686967166044,
    4.911148380271408, -2.242505081738489,
)
_LN2 = 0.6931471805599453


def _ln_12(x):
    acc = jnp.full((L,), _LN_COEFS[0], jnp.float32)
    for c in _LN_COEFS[1:]:
        acc = acc * x + jnp.float32(c)
    return acc


def _ln_pos(x):
    bits = plsc.bitcast(x, jnp.int32)
    e = ((bits >> 23) & 0xFF) - 127
    mant = plsc.bitcast((bits & 0x7FFFFF) | 0x3F800000, jnp.float32)
    return e.astype(jnp.float32) * jnp.float32(_LN2) + _ln_12(mant)


def _iota():
    return lax.broadcasted_iota(jnp.int32, (L,), 0)


def _walk(histsum, t):
    def body(c, carry):
        cnt, bacc, sacc = carry
        chunk = histsum[pl.ds(c * L, L)]
        cs = jnp.cumsum(chunk)
        below = cnt + cs - chunk
        is_b = (below < t) & (below + chunk >= t)
        bacc = bacc + jnp.where(is_b, c * L + _iota(), 0)
        sacc = sacc + jnp.where(is_b, below, 0)
        cnt = cnt + jnp.max(cs)
        return cnt, bacc, sacc

    zero = jnp.zeros((L,), jnp.int32)
    _, bacc, sacc = lax.fori_loop(0, HB // L, body,
                                  (jnp.int32(0), zero, zero))
    return jnp.sum(bacc), jnp.sum(sacc)


def _radix_round(r, wid, get_cand, sh, top, flip, pref, t,
                 hist2d, histsum, rdhist, sthist):
    zero = jnp.zeros((L,), jnp.int32)
    ones = jnp.ones((L,), jnp.int32)

    def zbody(i, _):
        hist2d[pl.ds(i * L, L)] = zero
        return 0
    lax.fori_loop(0, HB, zbody, 0)

    io = _iota()

    def scan(v, _):
        mask, key = get_cand(v)
        if not top:
            mask = mask & ((key >> (sh + 8)) == (pref >> (sh + 8)))
        bucket = (key >> sh) & (HB - 1)
        if flip:
            bucket = (HB - 1) - bucket
        slot = io * HB + bucket
        plsc.addupdate_scatter(hist2d, [slot], ones, mask=mask)
        return 0
    lax.fori_loop(0, NV, scan, 0)

    def lred_c(c, _):
        def lred_l(l, acc):
            return acc + hist2d[pl.ds(l * HB + c * L, L)]
        histsum[pl.ds(c * L, L)] = lax.fori_loop(0, L, lred_l, zero)
        return 0
    lax.fori_loop(0, HB // L, lred_c, 0)

    pltpu.sync_copy(histsum, sthist.at[pl.ds(r * NW * HB + wid * HB, HB)])
    plsc.subcore_barrier()
    pltpu.sync_copy(sthist.at[pl.ds(r * NW * HB, NW * HB)], rdhist)

    def gred_c(c, _):
        def gred_i(i, acc):
            return acc + rdhist[pl.ds(i * HB + c * L, L)]
        histsum[pl.ds(c * L, L)] = lax.fori_loop(0, NW, gred_i, zero)
        return 0
    lax.fori_loop(0, HB // L, gred_c, 0)

    bstar, taken = _walk(histsum, t)
    if flip:
        bstar = (HB - 1) - bstar
    return pref | (bstar << sh), t - taken


_GO_Y0 = 0
_GO_X0 = G * L
_GO_Y1 = 2 * G * L
_GO_X1 = 3 * G * L
_GO_AB = 4 * G * L
_GO_C0 = 5 * G * L
_GO_C1 = 5 * G * L + GP
_GO_C2 = 5 * G * L + 2 * GP
_GO_C3 = 5 * G * L + 3 * GP
_GO_HW = 5 * G * L + 4 * GP
GTB = _GO_HW + 2 * L


def _sc_kernel_body(ay0h, ax0h, ay1h, ax1h, s0h, s1h, p0h, p1h, p2h, p3h,
                    rkh, gtbh,
                    out_hbm,
                    cy0, cx0, cy1, cx1, vs0, vs1, vp0, vp1, vp2, vp3, vrkf,
                    gtb,
                    gtmax, maxiou, bestj, insd, areaa, forced, posm, negm,
                    poskey, hist2d, histsum, rdgt, rdhist, rdsm, wv, outv,
                    st_gtmax, st_cnt, st_hist, st_loss, dsem):
    wid = lax.axis_index("s")
    ostart = wid * NA
    gbase = jnp.minimum(ostart, N - NA)
    fzero = jnp.zeros((L,), jnp.float32)
    io = _iota()

    srcs = [ay0h, ax0h, ay1h, ax1h, s0h, s1h, p0h, p1h, p2h, p3h, rkh]
    dsts = [cy0, cx0, cy1, cx1, vs0, vs1, vp0, vp1, vp2, vp3, vrkf]
    copies = [
        pltpu.async_copy(src.at[pl.ds(gbase, NA)], dst, dsem)
        for src, dst in zip(srcs, dsts)
    ]
    copies.append(pltpu.async_copy(gtbh, gtb, dsem))
    for cp in copies:
        cp.wait()

    hvec = gtb[pl.ds(_GO_HW, L)]
    wvec = gtb[pl.ds(_GO_HW + L, L)]

    def coords(o):
        return (cy0[pl.ds(o, L)], cx0[pl.ds(o, L)],
                cy1[pl.ds(o, L)], cx1[pl.ds(o, L)])

    def p05(v, _):
        o = v * L
        a0, a1, a2, a3 = coords(o)
        own = (gbase + o + io) >= ostart
        ins = ((a0 >= 0.0) & (a1 >= 0.0) & (a2 <= hvec) & (a3 <= wvec)
               & own)
        insd[pl.ds(o, L)] = jnp.where(ins, 1.0, 0.0)
        areaa[pl.ds(o, L)] = (a2 - a0) * (a3 - a1)
        maxiou[pl.ds(o, L)] = fzero - 1e30
        bestj[pl.ds(o, L)] = jnp.zeros((L,), jnp.int32)
        forced[pl.ds(o, L)] = fzero
        return 0
    lax.fori_loop(0, NV, p05, 0)

    for b in range(G // JB):
        gd = []
        for jj in range(JB):
            j = b * JB + jj
            gd.append((gtb[pl.ds(_GO_Y0 + j * L, L)],
                       gtb[pl.ds(_GO_X0 + j * L, L)],
                       gtb[pl.ds(_GO_Y1 + j * L, L)],
                       gtb[pl.ds(_GO_X1 + j * L, L)],
                       gtb[pl.ds(_GO_AB + j * L, L)]))

        def p1(v, carry):
            gtm = list(carry[0])
            gti = list(carry[1])
            o = v * L
            a0, a1, a2, a3 = coords(o)
            ins = insd[pl.ds(o, L)] > 0.5
            area_a = areaa[pl.ds(o, L)]
            best = maxiou[pl.ds(o, L)]
            bj = bestj[pl.ds(o, L)]
            oio = o + io
            for jj in range(JB):
                g0, g1, g2, g3, ab = gd[jj]
                ih = jnp.maximum(jnp.minimum(a2, g2) - jnp.maximum(a0, g0),
                                 0.0)
                iw = jnp.maximum(jnp.minimum(a3, g3) - jnp.maximum(a1, g1),
                                 0.0)
                inter = ih * iw
                iou = inter / ((area_a + ab) - inter)
                iou = jnp.where(ins, iou, -1.0)
                upd = iou > best
                best = jnp.where(upd, iou, best)
                bj = jnp.where(upd, b * JB + jj, bj)
                upd2 = iou > gtm[jj]
                gtm[jj] = jnp.where(upd2, iou, gtm[jj])
                gti[jj] = jnp.where(upd2, oio, gti[jj])
            maxiou[pl.ds(o, L)] = best
            bestj[pl.ds(o, L)] = bj
            return tuple(gtm), tuple(gti)

        init = (tuple(fzero - 1e30 for _ in range(JB)),
                tuple(jnp.zeros((L,), jnp.int32) for _ in range(JB)))
        gtm, gti = lax.fori_loop(0, NV, p1, init)
        for jj in range(JB):
            j = b * JB + jj
            gtmax[pl.ds(j * L, L)] = gtm[jj]
            rdgt[pl.ds(j * L, L)] = gti[jj].astype(jnp.float32)

    pltpu.sync_copy(gtmax, st_gtmax.at[pl.ds(wid * G * L, G * L)])
    plsc.subcore_barrier()
    pltpu.sync_copy(st_gtmax, rdgt.at[pl.ds(G * L, NW * G * L)])

    onesf = fzero + 1.0
    for j in range(G):
        def fmax(i, acc):
            return jnp.maximum(
                acc, rdgt[pl.ds(G * L + i * G * L + j * L, L)])
        gm = lax.fori_loop(0, NW, fmax, fzero - 1e30)
        g = jnp.max(gm)
        mine = gtmax[pl.ds(j * L, L)]
        match = (mine >= g) & (mine > 0.0)
        cidx = rdgt[pl.ds(j * L, L)].astype(jnp.int32)
        plsc.store_scatter(forced, [cidx], onesf, mask=match)

    def p2_body(v, carry):
        cp_acc, cn_acc = carry
        o = v * L
        best = maxiou[pl.ds(o, L)]
        ins = insd[pl.ds(o, L)] > 0.5
        fc = forced[pl.ds(o, L)] > 0.5
        pm = ins & ((best >= POS_T) | fc)
        nm = ins & (best < NEG_T) & (best >= 0.0)
        posm[pl.ds(o, L)] = jnp.where(pm, 1.0, 0.0)
        negm[pl.ds(o, L)] = jnp.where(nm, 1.0, 0.0)
        key = plsc.bitcast(best, jnp.int32)
        poskey[pl.ds(o, L)] = jnp.where(pm, key, 0)
        return (cp_acc + jnp.where(pm, 1.0, 0.0),
                cn_acc + jnp.where(nm, 1.0, 0.0))

    cp_acc, cn_acc = lax.fori_loop(0, NV, p2_body, (fzero, fzero))
    wv[pl.ds(0, L)] = cp_acc
    wv[pl.ds(L, L)] = cn_acc
    pltpu.sync_copy(wv, st_cnt.at[pl.ds(wid * 2 * L, 2 * L)])
    plsc.subcore_barrier()
    pltpu.sync_copy(st_cnt, rdsm)

    def cmerge(i, carry):
        a, b2 = carry
        return (a + rdsm[pl.ds(i * 2 * L, L)],
                b2 + rdsm[pl.ds(i * 2 * L + L, L)])
    cpv, cnv = lax.fori_loop(0, NW, cmerge, (fzero, fzero))
    cnt_pos = jnp.sum(cpv)
    cnt_neg = jnp.sum(cnv)
    n_pos = jnp.minimum(cnt_pos, float(MAX_POS))
    need_f = float(TOTAL) - n_pos
    cp_i = cnt_pos.astype(jnp.int32)
    cn_i = cnt_neg.astype(jnp.int32)
    need_i = jnp.int32(TOTAL) - jnp.minimum(cp_i, MAX_POS)
    kfill_i = jnp.maximum(need_i - cn_i, 0)
    pos_over = cnt_pos > float(MAX_POS)
    neg_over = cnt_neg > need_f

    def get_pos(v):
        o = v * L
        return posm[pl.ds(o, L)] > 0.5, poskey[pl.ds(o, L)]

    def get_neg(v):
        o = v * L
        return (negm[pl.ds(o, L)] > 0.5,
                plsc.bitcast(vrkf[pl.ds(o, L)], jnp.int32))

    def rrnd(r, get, sh, top, flip, pref, t):
        return _radix_round(r, wid, get, sh, top, flip, pref, t,
                            hist2d, histsum, rdhist, st_hist)

    def pos_rounds(_):
        pref, t = rrnd(0, get_pos, 24, True, True, jnp.int32(0),
                       jnp.int32(MAX_POS))
        pref, t = rrnd(1, get_pos, 16, False, True, pref, t)
        pref, t = rrnd(2, get_pos, 8, False, True, pref, t)
        pref, t = rrnd(3, get_pos, 0, False, True, pref, t)
        kv_, trem = pref, t

        def get_tie(v):
            o = v * L
            m = (posm[pl.ds(o, L)] > 0.5) & (poskey[pl.ds(o, L)] == kv_)
            return m, gbase + o + io

        pref2, t2 = rrnd(4, get_tie, 8, True, False, jnp.int32(0), trem)
        pref2, _ = rrnd(5, get_tie, 0, False, False, pref2, t2)
        return kv_, pref2

    kv, ki = lax.cond(pos_over, pos_rounds,
                      lambda _: (jnp.int32(0), jnp.int32(NPAD)),
                      0)

    def neg_rounds(_):
        pref3, t3 = rrnd(6, get_neg, 8, True, False, jnp.int32(0), need_i)
        pref3, _ = rrnd(7, get_neg, 0, False, False, pref3, t3)
        return pref3

    kr = lax.cond(neg_over, neg_rounds, lambda _: jnp.int32(NPAD + 1), 0)

    def get_fill(v):
        o = v * L
        idx = gbase + o + io
        m = (negm[pl.ds(o, L)] <= 0.5) & (idx >= ostart)
        return m, idx

    def fill_rounds(_):
        pref4, t4 = rrnd(8, get_fill, 8, True, False, jnp.int32(0), kfill_i)
        pref4, _ = rrnd(9, get_fill, 0, False, False, pref4, t4)
        return pref4

    kf = lax.cond(kfill_i > 0, fill_rounds, lambda _: jnp.int32(-1), 0)

    def p4_body(v, carry):
        acc_cls, acc_reg = carry
        o = v * L
        idx = gbase + o + io
        pm = posm[pl.ds(o, L)] > 0.5
        nm = negm[pl.ds(o, L)] > 0.5
        key = poskey[pl.ds(o, L)]
        rk = plsc.bitcast(vrkf[pl.ds(o, L)], jnp.int32)
        sel_pos = pm & ((key > kv) | ((key == kv) & (idx <= ki)))
        sel_neg = nm & (rk <= kr)
        sel_fill = (~nm) & (idx >= ostart) & (idx <= kf)
        neg_w = sel_neg | sel_fill

        sv0 = vs0[pl.ds(o, L)]
        sv1 = vs1[pl.ds(o, L)]
        m = jnp.maximum(sv0, sv1)
        esum = jnp.exp(sv0 - m) + jnp.exp(sv1 - m)
        lse = m + _ln_12(esum)
        lp0 = sv0 - lse
        lp1 = sv1 - lse
        acc_cls = (acc_cls - jnp.where(sel_pos, lp1, 0.0)
                   - jnp.where(neg_w, lp0, 0.0))

        a0, a1, a2, a3 = coords(o)
        bj = bestj[pl.ds(o, L)]
        g0 = plsc.load_gather(gtb, [_GO_C0 + bj])
        g1 = plsc.load_gather(gtb, [_GO_C1 + bj])
        g2 = plsc.load_gather(gtb, [_GO_C2 + bj])
        g3 = plsc.load_gather(gtb, [_GO_C3 + bj])
        a_h = a2 - a0
        a_w = a3 - a1
        a_cy = a0 + 0.5 * a_h
        a_cx = a1 + 0.5 * a_w
        g_h = g2 - g0
        g_w = g3 - g1
        g_cy = g0 + 0.5 * g_h
        g_cx = g1 + 0.5 * g_w
        eps = jnp.float32(1e-8)
        ty = (g_cy - a_cy) / (a_h + eps)
        tx = (g_cx - a_cx) / (a_w + eps)
        th = _ln_pos(jnp.maximum(g_h, eps) / jnp.maximum(a_h, eps))
        tw = _ln_pos(jnp.maximum(g_w, eps) / jnp.maximum(a_w, eps))
        ssum = fzero
        for pref_ref, tgt in ((vp0, tx), (vp1, ty), (vp2, tw), (vp3, th)):
            d = pref_ref[pl.ds(o, L)] - tgt
            ad = jnp.abs(d)
            sl = jnp.where(ad < 1.0 / SIG2, 0.5 * SIG2 * d * d,
                           ad - 0.5 / SIG2)
            ssum = ssum + sl
        acc_reg = acc_reg + jnp.where(sel_pos, ssum, 0.0)
        return acc_cls, acc_reg

    acc_cls, acc_reg = lax.fori_loop(0, NV, p4_body, (fzero, fzero))
    wv[pl.ds(0, L)] = acc_cls
    wv[pl.ds(L, L)] = acc_reg
    pltpu.sync_copy(wv, st_loss.at[pl.ds(wid * 2 * L, 2 * L)])
    plsc.subcore_barrier()
    pltpu.sync_copy(st_loss, rdsm)

    def lmerge(i, carry):
        a, b2 = carry
        return (a + rdsm[pl.ds(i * 2 * L, L)],
                b2 + rdsm[pl.ds(i * 2 * L + L, L)])
    av, bv = lax.fori_loop(0, NW, lmerge, (fzero, fzero))
    cls_sum = jnp.sum(av)
    reg_sum = jnp.sum(bv)
    numer = (jnp.where(io == 0, cls_sum, 0.0)
             + jnp.where(io == 1, reg_sum, 0.0))
    denom = jnp.where(io == 1, jnp.maximum(n_pos, 1.0),
                      jnp.float32(TOTAL))
    outv[pl.ds(0, L)] = numer / denom

    @pl.when(wid == 0)
    def _():
        pltpu.sync_copy(outv, out_hbm)


def _build_call():
    mesh = plsc.VectorSubcoreMesh(core_axis_name="c", subcore_axis_name="s",
                                  num_cores=1, num_subcores=NW)
    f32, i32 = jnp.float32, jnp.int32
    return pl.kernel(
        _sc_kernel_body,
        out_type=[
            jax.ShapeDtypeStruct((L,), f32),
        ],
        mesh=mesh,
        compiler_params=pltpu.CompilerParams(needs_layout_passes=False),
        scratch_types=[
            pltpu.VMEM((NA,), f32),
            pltpu.VMEM((NA,), f32),
            pltpu.VMEM((NA,), f32),
            pltpu.VMEM((NA,), f32),
            pltpu.VMEM((NA,), f32),
            pltpu.VMEM((NA,), f32),
            pltpu.VMEM((NA,), f32),
            pltpu.VMEM((NA,), f32),
            pltpu.VMEM((NA,), f32),
            pltpu.VMEM((NA,), f32),
            pltpu.VMEM((NA,), f32),
            pltpu.VMEM((GTB,), f32),
            pltpu.VMEM((G * L,), f32),
            pltpu.VMEM((NA,), f32),
            pltpu.VMEM((NA,), i32),
            pltpu.VMEM((NA,), f32),
            pltpu.VMEM((NA,), f32),
            pltpu.VMEM((NA,), f32),
            pltpu.VMEM((NA,), f32),
            pltpu.VMEM((NA,), f32),
            pltpu.VMEM((NA,), i32),
            pltpu.VMEM((L * HB,), i32),
            pltpu.VMEM((HB,), i32),
            pltpu.VMEM(((NW + 1) * G * L,), f32),
            pltpu.VMEM((NW * HB,), i32),
            pltpu.VMEM((NW * 2 * L,), f32),
            pltpu.VMEM((2 * L,), f32),
            pltpu.VMEM((L,), f32),
            pltpu.VMEM_SHARED((NW * G * L,), f32),
            pltpu.VMEM_SHARED((NW * 2 * L,), f32),
            pltpu.VMEM_SHARED((NR * NW * HB,), i32),
            pltpu.VMEM_SHARED((NW * 2 * L,), f32),
            pltpu.SemaphoreType.DMA,
        ],
    )


def kernel(image_shape, anchors, rpn_score, rpn_bboxes_txtytwth, gt_bboxes):
    f32 = jnp.float32
    with jax.ensure_compile_time_eval():
        rngv = jax.random.uniform(jax.random.key(123), (N,))
        order = jnp.argsort(-rngv, stable=True)
        rank = jnp.argsort(order, stable=True).astype(jnp.int32)
        rkf = lax.bitcast_convert_type(rank, f32)

    a = anchors.astype(f32)
    s = rpn_score.astype(f32)
    p = rpn_bboxes_txtytwth.astype(f32)

    gt = gt_bboxes.astype(f32)
    ab = ((gt[:, 2] - gt[:, 0]) * (gt[:, 3] - gt[:, 1]) + 1e-9)
    gpad = jnp.zeros((GP - G,), f32)
    gtbuf = jnp.concatenate([
        jnp.broadcast_to(gt[:, 0:1], (G, L)).reshape(-1),
        jnp.broadcast_to(gt[:, 1:2], (G, L)).reshape(-1),
        jnp.broadcast_to(gt[:, 2:3], (G, L)).reshape(-1),
        jnp.broadcast_to(gt[:, 3:4], (G, L)).reshape(-1),
        jnp.broadcast_to(ab[:, None], (G, L)).reshape(-1),
        gt[:, 0], gpad, gt[:, 1], gpad, gt[:, 2], gpad, gt[:, 3], gpad,
        jnp.full((L,), image_shape[0], f32),
        jnp.full((L,), image_shape[1], f32),
    ])

    call = _build_call()
    out = call(a[:, 0], a[:, 1], a[:, 2], a[:, 3], s[:, 0], s[:, 1],
               p[:, 0], p[:, 1], p[:, 2], p[:, 3], rkf, gtbuf)[0]
    return (out[0], out[1])
